# baseline (speedup 1.0000x reference)
import functools

import jax
import jax.numpy as jnp
from jax import lax
from jax.experimental import pallas as pl
from jax.experimental.pallas import tpu as pltpu
from jax.experimental.pallas import tpu_sc as plsc

N = 10000
E = 320000
D = 128
DE = 16
G = 64
OUT = 64

NC = 2
NS = 16
NW = NC * NS

EB = 64
IC = 32
BPW0 = 160
BPW1 = 160
NB0 = NS * BPW0
E_PAD = NS * (BPW0 + BPW1) * EB
ACC_ROWS = 10240
RPT = ACC_ROWS // NS

_sc_mesh = plsc.VectorSubcoreMesh(core_axis_name="c", subcore_axis_name="s")


@functools.partial(
    pl.kernel,
    out_type=jax.ShapeDtypeStruct((NC, ACC_ROWS, D), jnp.float32),
    mesh=_sc_mesh,
    scratch_types=[
        pltpu.VMEM((IC, EB), jnp.int32),
        pltpu.VMEM((IC, EB), jnp.int32),
        pltpu.VMEM((EB, D), jnp.float32),
        pltpu.VMEM((EB, D), jnp.float32),
        pltpu.VMEM((EB // 8, 8 * D), jnp.float32),
        pltpu.VMEM((EB // 8, 8 * D), jnp.float32),
        pltpu.VMEM_SHARED((ACC_ROWS, D), jnp.float32),
        pltpu.SemaphoreType.DMA,
        pltpu.SemaphoreType.DMA,
        pltpu.SemaphoreType.DMA,
        pltpu.SemaphoreType.DMA,
    ],
)
def _sc_aggregate(h_hbm, el_hbm, er_hbm, src_hbm, dst_hbm, out_hbm,
                  src_v, dst_v, rows0_v, rows1_v, ev0_v, ev1_v,
                  acc_sh, gsem0, gsem1, esem0, esem1):
    c = lax.axis_index("c")
    s = lax.axis_index("s")
    bpw = jnp.where(c == 0, BPW0, BPW1)
    wbase = jnp.where(c == 0, s * BPW0, NB0 + s * BPW1)

    rows_b = (rows0_v, rows1_v)
    ev_b = (ev0_v, ev1_v)
    gsem_b = (gsem0, gsem1)
    esem_b = (esem0, esem1)

    def zrow(r, carry):
        for kk in range(D // 16):
            rows0_v[r, pl.ds(kk * 16, 16)] = jnp.zeros((16,), jnp.float32)
        return carry

    lax.fori_loop(0, EB, zrow, 0)

    def zcopy(r, carry):
        pltpu.sync_copy(rows0_v, acc_sh.at[pl.ds(s * RPT + r * EB, EB)])
        return carry

    lax.fori_loop(0, RPT // EB, zcopy, 0)
    plsc.subcore_barrier()

    def chunk(cc, carry):
        base = wbase + cc * IC
        pltpu.sync_copy(src_hbm.at[pl.ds(base, IC)], src_v)
        pltpu.sync_copy(dst_hbm.at[pl.ds(base, IC)], dst_v)

        e_arr = (el_hbm, er_hbm)

        def issue(jj, b):
            pltpu.async_copy(h_hbm.at[src_v.at[jj]], rows_b[b], gsem_b[b])
            pltpu.async_copy(
                e_arr[b].at[pl.ds((base + jj) // 2 * (EB // 8), EB // 8)],
                ev_b[b], esem_b[b])

        issue(0, 0)

        def pair(p, carry1):
            for b in range(2):
                j = p * 2 + b
                nb = 1 - b

                @pl.when(j + 1 < IC)
                def _():
                    issue(j + 1, nb)

                pltpu.make_async_copy(
                    h_hbm.at[src_v.at[j]], rows_b[b], gsem_b[b]).wait()
                pltpu.make_async_copy(
                    e_arr[b].at[pl.ds((base + j) // 2 * (EB // 8), EB // 8)],
                    ev_b[b], esem_b[b]).wait()

                rv, ev = rows_b[b], ev_b[b]

                def elem(r, carry2):
                    rr = r // 8
                    cb = (r % 8) * D
                    for kk in range(D // 16):
                        v = rv[r, pl.ds(kk * 16, 16)] + ev[rr, pl.ds(cb + kk * 16, 16)]
                        rv[r, pl.ds(kk * 16, 16)] = jnp.maximum(v, 0.0)
                    return carry2

                lax.fori_loop(0, EB, elem, 0)
                pltpu.sync_copy(rv, acc_sh.at[dst_v.at[j]], add=True)
            return carry1

        lax.fori_loop(0, IC // 2, pair, 0)
        return carry

    lax.fori_loop(0, bpw // IC, chunk, 0)
    plsc.subcore_barrier()
    pltpu.sync_copy(acc_sh.at[pl.ds(s * RPT, RPT)],
                    out_hbm.at[c, pl.ds(s * RPT, RPT)])


_E_RB8 = 320
_E_NB8 = E_PAD // 8 // _E_RB8


def _split_eo8(e):
    e3 = e.reshape(_E_RB8 // 16, 16, 8 * D)
    ev = e3[:, :8, :].reshape(_E_RB8 // 2, 8 * D)
    od = e3[:, 8:, :].reshape(_E_RB8 // 2, 8 * D)
    return ev, od


def _edge_mlp1_body(ea_ref, W8_ref, be8_ref, el_ref, er_ref):
    e = (jnp.dot(ea_ref[...], W8_ref[...],
                 preferred_element_type=jnp.float32) + be8_ref[...])
    el_ref[...], er_ref[...] = _split_eo8(e)


def _edge_mlp2_body(ea_ref, W81_ref, be81_ref, W82_ref, be82_ref,
                    e1l_ref, e1r_ref, e2l_ref, e2r_ref):
    a = ea_ref[...]
    e1 = jnp.dot(a, W81_ref[...], preferred_element_type=jnp.float32) + be81_ref[...]
    e1l_ref[...], e1r_ref[...] = _split_eo8(e1)
    e2 = jnp.dot(a, W82_ref[...], preferred_element_type=jnp.float32) + be82_ref[...]
    e2l_ref[...], e2r_ref[...] = _split_eo8(e2)


_E_WSPEC = pl.BlockSpec((D, 8 * D), lambda i: (0, 0))
_E_BSPEC = pl.BlockSpec((1, 8 * D), lambda i: (0, 0))
_E_ASPEC = pl.BlockSpec((_E_RB8, D), lambda i: (i, 0))
_E_OSPEC = pl.BlockSpec((_E_RB8 // 2, 8 * D), lambda i: (i, 0))
_E_OSHAPE = jax.ShapeDtypeStruct((E_PAD // 16, 8 * D), jnp.float32)


def _edge_mlp1(ea8, W8, be8):
    return pl.pallas_call(
        _edge_mlp1_body,
        grid=(E_PAD // 8 // _E_RB8,),
        in_specs=[_E_ASPEC, _E_WSPEC, _E_BSPEC],
        out_specs=[_E_OSPEC, _E_OSPEC],
        out_shape=[_E_OSHAPE, _E_OSHAPE],
    )(ea8, W8, be8)


def _edge_mlp2(ea8, W81, be81, W82, be82):
    return pl.pallas_call(
        _edge_mlp2_body,
        grid=(E_PAD // 8 // _E_RB8,),
        in_specs=[_E_ASPEC, _E_WSPEC, _E_BSPEC, _E_WSPEC, _E_BSPEC],
        out_specs=[_E_OSPEC] * 4,
        out_shape=[_E_OSHAPE] * 4,
    )(ea8, W81, be81, W82, be82)


_N_RB = 400


def _update_body(h_ref, p0_ref, p1_ref, Wn_ref, bn_ref, o_ref):
    hs = h_ref[...] + p0_ref[0] + p1_ref[0]
    o_ref[...] = jnp.maximum(
        jnp.dot(hs, Wn_ref[...], preferred_element_type=jnp.float32) + bn_ref[...],
        0.0)


def _update(h, parts, Wn, bn):
    grid = (N // _N_RB,)
    return pl.pallas_call(
        _update_body,
        grid=grid,
        in_specs=[
            pl.BlockSpec((_N_RB, D), lambda i: (i, 0)),
            pl.BlockSpec((1, _N_RB, D), lambda i: (0, i, 0)),
            pl.BlockSpec((1, _N_RB, D), lambda i: (1, i, 0)),
            pl.BlockSpec((D, D), lambda i: (0, 0)),
            pl.BlockSpec((1, D), lambda i: (0, 0)),
        ],
        out_specs=pl.BlockSpec((_N_RB, D), lambda i: (i, 0)),
        out_shape=jax.ShapeDtypeStruct((N, D), jnp.float32),
    )(h, parts, parts, Wn, bn)


_P_RB = 400
_P_NB = N // _P_RB


def _upool_body(h_ref, p0_ref, p1_ref, Wn_ref, bn_ref, bcol_ref,
                W1_ref, b1_ref, W2_ref, b2_ref, o_ref,
                maxs_ref, sums_ref, cnt_ref):
    i = pl.program_id(0)

    @pl.when(i == 0)
    def _():
        maxs_ref[...] = jnp.full((G, D), -1e30, jnp.float32)
        sums_ref[...] = jnp.zeros((G, D), jnp.float32)
        cnt_ref[...] = jnp.zeros((G, 1), jnp.float32)

    hs = h_ref[...] + p0_ref[0] + p1_ref[0]
    h3 = jnp.maximum(
        jnp.dot(hs, Wn_ref[...], preferred_element_type=jnp.float32) + bn_ref[...],
        0.0)

    bcol = bcol_ref[...]
    gmin = jnp.min(bcol)
    gmax = jnp.max(bcol)

    def gbody(g, carry):
        mask = bcol == g
        m = jnp.max(jnp.where(mask, h3, -1e30), axis=0, keepdims=True)
        maxs_ref[pl.ds(g, 1), :] = jnp.maximum(maxs_ref[pl.ds(g, 1), :], m)
        s = jnp.sum(jnp.where(mask, h3, 0.0), axis=0, keepdims=True)
        sums_ref[pl.ds(g, 1), :] += s
        cnt_ref[pl.ds(g, 1), :] += jnp.sum(
            mask.astype(jnp.float32), axis=0, keepdims=True)
        return carry

    lax.fori_loop(gmin, gmax + 1, gbody, 0)

    @pl.when(i == _P_NB - 1)
    def _():
        counts = cnt_ref[...]
        mean = sums_ref[...] / jnp.maximum(counts, 1.0)
        maxs = jnp.where(counts > 0, maxs_ref[...], 0.0)
        gf = jnp.concatenate([maxs, mean], axis=1)
        hid = jnp.maximum(
            jnp.dot(gf, W1_ref[...], preferred_element_type=jnp.float32) + b1_ref[...],
            0.0)
        o_ref[...] = jnp.dot(hid, W2_ref[...],
                             preferred_element_type=jnp.float32) + b2_ref[...]


def _update_pool(h, parts, Wn, bn, bcol, W1, b1, W2, b2):
    return pl.pallas_call(
        _upool_body,
        grid=(_P_NB,),
        in_specs=[
            pl.BlockSpec((_P_RB, D), lambda i: (i, 0)),
            pl.BlockSpec((1, _P_RB, D), lambda i: (0, i, 0)),
            pl.BlockSpec((1, _P_RB, D), lambda i: (1, i, 0)),
            pl.BlockSpec((D, D), lambda i: (0, 0)),
            pl.BlockSpec((1, D), lambda i: (0, 0)),
            pl.BlockSpec((_P_RB, 1), lambda i: (i, 0)),
            pl.BlockSpec((2 * D, D // 2), lambda i: (0, 0)),
            pl.BlockSpec((1, D // 2), lambda i: (0, 0)),
            pl.BlockSpec((D // 2, OUT), lambda i: (0, 0)),
            pl.BlockSpec((1, OUT), lambda i: (0, 0)),
        ],
        out_specs=pl.BlockSpec((G, OUT), lambda i: (0, 0)),
        out_shape=jax.ShapeDtypeStruct((G, OUT), jnp.float32),
        scratch_shapes=[pltpu.VMEM((G, D), jnp.float32),
                        pltpu.VMEM((G, D), jnp.float32),
                        pltpu.VMEM((G, 1), jnp.float32)],
    )(h, parts, parts, Wn, bn, bcol, W1, b1, W2, b2)


def kernel(x, edge_index, edge_attr, batch,
           We0, be0, Wn0, bn0, We1, be1, Wn1, bn1, We2, be2, Wn2, bn2,
           W1, b1, W2, b2):
    pad = E_PAD - E
    src = jnp.concatenate(
        [edge_index[0], jnp.arange(pad, dtype=jnp.int32) % N])
    dst = jnp.concatenate(
        [edge_index[1], N + (jnp.arange(pad, dtype=jnp.int32) % (ACC_ROWS - N))])
    srcp = src.reshape(E_PAD // EB, EB)
    dstp = dst.reshape(E_PAD // EB, EB)

    ea8 = edge_attr.reshape(E // 8, D)

    def blockdiag8(We):
        z = jnp.zeros((8, DE, 8, D), jnp.float32)
        z = z.at[jnp.arange(8), :, jnp.arange(8), :].set(We)
        return z.reshape(8 * DE, 8 * D)

    def tile8(be):
        return jnp.tile(be, (8,)).reshape(1, 8 * D)

    e0l, e0r = _edge_mlp1(ea8, blockdiag8(We0), tile8(be0))
    parts = _sc_aggregate(x, e0l, e0r, srcp, dstp)
    e1l, e1r, e2l, e2r = _edge_mlp2(ea8, blockdiag8(We1), tile8(be1),
                                    blockdiag8(We2), tile8(be2))
    h = _update(x, parts, Wn0, bn0.reshape(1, D))

    parts = _sc_aggregate(h, e1l, e1r, srcp, dstp)
    h = _update(h, parts, Wn1, bn1.reshape(1, D))

    parts = _sc_aggregate(h, e2l, e2r, srcp, dstp)
    return _update_pool(h, parts, Wn2, bn2.reshape(1, D), batch.reshape(N, 1),
                        W1, b1.reshape(1, D // 2), W2, b2.reshape(1, OUT))


# speedup vs baseline: 1.0070x; 1.0070x over previous
import functools

import jax
import jax.numpy as jnp
from jax import lax
from jax.experimental import pallas as pl
from jax.experimental.pallas import tpu as pltpu
from jax.experimental.pallas import tpu_sc as plsc

N = 10000
E = 320000
D = 128
DE = 16
G = 64
OUT = 64

NC = 2
NS = 16
NW = NC * NS

EB = 64
IC = 32
BPW0 = 160
BPW1 = 160
NB0 = NS * BPW0
E_PAD = NS * (BPW0 + BPW1) * EB
ACC_ROWS = 10240
RPT = ACC_ROWS // NS

_sc_mesh = plsc.VectorSubcoreMesh(core_axis_name="c", subcore_axis_name="s")


@functools.partial(
    pl.kernel,
    out_type=jax.ShapeDtypeStruct((NC, ACC_ROWS, D), jnp.float32),
    mesh=_sc_mesh,
    scratch_types=[
        pltpu.VMEM((IC, EB), jnp.int32),
        pltpu.VMEM((IC, EB), jnp.int32),
        pltpu.VMEM((EB, D), jnp.float32),
        pltpu.VMEM((EB, D), jnp.float32),
        pltpu.VMEM((EB // 8, 8 * D), jnp.float32),
        pltpu.VMEM((EB // 8, 8 * D), jnp.float32),
        pltpu.VMEM_SHARED((ACC_ROWS, D), jnp.float32),
        pltpu.SemaphoreType.DMA,
        pltpu.SemaphoreType.DMA,
        pltpu.SemaphoreType.DMA,
        pltpu.SemaphoreType.DMA,
    ],
)
def _sc_aggregate(h_hbm, el_hbm, er_hbm, src_hbm, dst_hbm, out_hbm,
                  src_v, dst_v, rows0_v, rows1_v, ev0_v, ev1_v,
                  acc_sh, gsem0, gsem1, esem0, esem1):
    c = lax.axis_index("c")
    s = lax.axis_index("s")
    bpw = jnp.where(c == 0, BPW0, BPW1)
    wbase = jnp.where(c == 0, s * BPW0, NB0 + s * BPW1)

    rows_b = (rows0_v, rows1_v)
    ev_b = (ev0_v, ev1_v)
    gsem_b = (gsem0, gsem1)
    esem_b = (esem0, esem1)

    def zrow(r, carry):
        for kk in range(D // 16):
            rows0_v[r, pl.ds(kk * 16, 16)] = jnp.zeros((16,), jnp.float32)
        return carry

    lax.fori_loop(0, EB, zrow, 0)

    def zcopy(r, carry):
        pltpu.sync_copy(rows0_v, acc_sh.at[pl.ds(s * RPT + r * EB, EB)])
        return carry

    lax.fori_loop(0, RPT // EB, zcopy, 0)
    plsc.subcore_barrier()

    def chunk(cc, carry):
        base = wbase + cc * IC
        pltpu.sync_copy(src_hbm.at[pl.ds(base, IC)], src_v)
        pltpu.sync_copy(dst_hbm.at[pl.ds(base, IC)], dst_v)

        e_arr = (el_hbm, er_hbm)

        def issue(jj, b):
            pltpu.async_copy(h_hbm.at[src_v.at[jj]], rows_b[b], gsem_b[b])
            pltpu.async_copy(
                e_arr[b].at[pl.ds((base + jj) // 2 * (EB // 8), EB // 8)],
                ev_b[b], esem_b[b])

        issue(0, 0)

        def pair(p, carry1):
            for b in range(2):
                j = p * 2 + b
                nb = 1 - b

                @pl.when(j + 1 < IC)
                def _():
                    issue(j + 1, nb)

                pltpu.make_async_copy(
                    h_hbm.at[src_v.at[j]], rows_b[b], gsem_b[b]).wait()
                pltpu.make_async_copy(
                    e_arr[b].at[pl.ds((base + j) // 2 * (EB // 8), EB // 8)],
                    ev_b[b], esem_b[b]).wait()

                rv, ev = rows_b[b], ev_b[b]

                def elem(rr, carry2):
                    r0 = rr * 8
                    for jj in range(8):
                        for kk in range(D // 16):
                            sl = pl.ds(kk * 16, 16)
                            esl = pl.ds(jj * D + kk * 16, 16)
                            v = rv[r0 + jj, sl] + ev[rr, esl]
                            rv[r0 + jj, sl] = jnp.maximum(v, 0.0)
                    return carry2

                lax.fori_loop(0, EB // 8, elem, 0)
                pltpu.sync_copy(rv, acc_sh.at[dst_v.at[j]], add=True)
            return carry1

        lax.fori_loop(0, IC // 2, pair, 0)
        return carry

    lax.fori_loop(0, bpw // IC, chunk, 0)
    plsc.subcore_barrier()
    pltpu.sync_copy(acc_sh.at[pl.ds(s * RPT, RPT)],
                    out_hbm.at[c, pl.ds(s * RPT, RPT)])


_E_RB8 = 320
_E_NB8 = E_PAD // 8 // _E_RB8


def _split_eo8(e):
    e3 = e.reshape(_E_RB8 // 16, 16, 8 * D)
    ev = e3[:, :8, :].reshape(_E_RB8 // 2, 8 * D)
    od = e3[:, 8:, :].reshape(_E_RB8 // 2, 8 * D)
    return ev, od


def _edge_mlp1_body(ea_ref, W8_ref, be8_ref, el_ref, er_ref):
    e = (jnp.dot(ea_ref[...], W8_ref[...],
                 preferred_element_type=jnp.float32) + be8_ref[...])
    el_ref[...], er_ref[...] = _split_eo8(e)


def _edge_mlp2_body(ea_ref, W81_ref, be81_ref, W82_ref, be82_ref,
                    e1l_ref, e1r_ref, e2l_ref, e2r_ref):
    a = ea_ref[...]
    e1 = jnp.dot(a, W81_ref[...], preferred_element_type=jnp.float32) + be81_ref[...]
    e1l_ref[...], e1r_ref[...] = _split_eo8(e1)
    e2 = jnp.dot(a, W82_ref[...], preferred_element_type=jnp.float32) + be82_ref[...]
    e2l_ref[...], e2r_ref[...] = _split_eo8(e2)


_E_WSPEC = pl.BlockSpec((D, 8 * D), lambda i: (0, 0))
_E_BSPEC = pl.BlockSpec((1, 8 * D), lambda i: (0, 0))
_E_ASPEC = pl.BlockSpec((_E_RB8, D), lambda i: (i, 0))
_E_OSPEC = pl.BlockSpec((_E_RB8 // 2, 8 * D), lambda i: (i, 0))
_E_OSHAPE = jax.ShapeDtypeStruct((E_PAD // 16, 8 * D), jnp.float32)


def _edge_mlp1(ea8, W8, be8):
    return pl.pallas_call(
        _edge_mlp1_body,
        grid=(E_PAD // 8 // _E_RB8,),
        in_specs=[_E_ASPEC, _E_WSPEC, _E_BSPEC],
        out_specs=[_E_OSPEC, _E_OSPEC],
        out_shape=[_E_OSHAPE, _E_OSHAPE],
    )(ea8, W8, be8)


def _edge_mlp2(ea8, W81, be81, W82, be82):
    return pl.pallas_call(
        _edge_mlp2_body,
        grid=(E_PAD // 8 // _E_RB8,),
        in_specs=[_E_ASPEC, _E_WSPEC, _E_BSPEC, _E_WSPEC, _E_BSPEC],
        out_specs=[_E_OSPEC] * 4,
        out_shape=[_E_OSHAPE] * 4,
    )(ea8, W81, be81, W82, be82)


_N_RB = 400


def _update_body(h_ref, p0_ref, p1_ref, Wn_ref, bn_ref, o_ref):
    hs = h_ref[...] + p0_ref[0] + p1_ref[0]
    o_ref[...] = jnp.maximum(
        jnp.dot(hs, Wn_ref[...], preferred_element_type=jnp.float32) + bn_ref[...],
        0.0)


def _update(h, parts, Wn, bn):
    grid = (N // _N_RB,)
    return pl.pallas_call(
        _update_body,
        grid=grid,
        in_specs=[
            pl.BlockSpec((_N_RB, D), lambda i: (i, 0)),
            pl.BlockSpec((1, _N_RB, D), lambda i: (0, i, 0)),
            pl.BlockSpec((1, _N_RB, D), lambda i: (1, i, 0)),
            pl.BlockSpec((D, D), lambda i: (0, 0)),
            pl.BlockSpec((1, D), lambda i: (0, 0)),
        ],
        out_specs=pl.BlockSpec((_N_RB, D), lambda i: (i, 0)),
        out_shape=jax.ShapeDtypeStruct((N, D), jnp.float32),
    )(h, parts, parts, Wn, bn)


_P_RB = 400
_P_NB = N // _P_RB


def _upool_body(h_ref, p0_ref, p1_ref, Wn_ref, bn_ref, bcol_ref,
                W1_ref, b1_ref, W2_ref, b2_ref, o_ref,
                maxs_ref, sums_ref, cnt_ref):
    i = pl.program_id(0)

    @pl.when(i == 0)
    def _():
        maxs_ref[...] = jnp.full((G, D), -1e30, jnp.float32)
        sums_ref[...] = jnp.zeros((G, D), jnp.float32)
        cnt_ref[...] = jnp.zeros((G, 1), jnp.float32)

    hs = h_ref[...] + p0_ref[0] + p1_ref[0]
    h3 = jnp.maximum(
        jnp.dot(hs, Wn_ref[...], preferred_element_type=jnp.float32) + bn_ref[...],
        0.0)

    bcol = bcol_ref[...]
    gmin = jnp.min(bcol)
    gmax = jnp.max(bcol)

    def gbody(g, carry):
        mask = bcol == g
        m = jnp.max(jnp.where(mask, h3, -1e30), axis=0, keepdims=True)
        maxs_ref[pl.ds(g, 1), :] = jnp.maximum(maxs_ref[pl.ds(g, 1), :], m)
        s = jnp.sum(jnp.where(mask, h3, 0.0), axis=0, keepdims=True)
        sums_ref[pl.ds(g, 1), :] += s
        cnt_ref[pl.ds(g, 1), :] += jnp.sum(
            mask.astype(jnp.float32), axis=0, keepdims=True)
        return carry

    lax.fori_loop(gmin, gmax + 1, gbody, 0)

    @pl.when(i == _P_NB - 1)
    def _():
        counts = cnt_ref[...]
        mean = sums_ref[...] / jnp.maximum(counts, 1.0)
        maxs = jnp.where(counts > 0, maxs_ref[...], 0.0)
        gf = jnp.concatenate([maxs, mean], axis=1)
        hid = jnp.maximum(
            jnp.dot(gf, W1_ref[...], preferred_element_type=jnp.float32) + b1_ref[...],
            0.0)
        o_ref[...] = jnp.dot(hid, W2_ref[...],
                             preferred_element_type=jnp.float32) + b2_ref[...]


def _update_pool(h, parts, Wn, bn, bcol, W1, b1, W2, b2):
    return pl.pallas_call(
        _upool_body,
        grid=(_P_NB,),
        in_specs=[
            pl.BlockSpec((_P_RB, D), lambda i: (i, 0)),
            pl.BlockSpec((1, _P_RB, D), lambda i: (0, i, 0)),
            pl.BlockSpec((1, _P_RB, D), lambda i: (1, i, 0)),
            pl.BlockSpec((D, D), lambda i: (0, 0)),
            pl.BlockSpec((1, D), lambda i: (0, 0)),
            pl.BlockSpec((_P_RB, 1), lambda i: (i, 0)),
            pl.BlockSpec((2 * D, D // 2), lambda i: (0, 0)),
            pl.BlockSpec((1, D // 2), lambda i: (0, 0)),
            pl.BlockSpec((D // 2, OUT), lambda i: (0, 0)),
            pl.BlockSpec((1, OUT), lambda i: (0, 0)),
        ],
        out_specs=pl.BlockSpec((G, OUT), lambda i: (0, 0)),
        out_shape=jax.ShapeDtypeStruct((G, OUT), jnp.float32),
        scratch_shapes=[pltpu.VMEM((G, D), jnp.float32),
                        pltpu.VMEM((G, D), jnp.float32),
                        pltpu.VMEM((G, 1), jnp.float32)],
    )(h, parts, parts, Wn, bn, bcol, W1, b1, W2, b2)


def kernel(x, edge_index, edge_attr, batch,
           We0, be0, Wn0, bn0, We1, be1, Wn1, bn1, We2, be2, Wn2, bn2,
           W1, b1, W2, b2):
    pad = E_PAD - E
    src = jnp.concatenate(
        [edge_index[0], jnp.arange(pad, dtype=jnp.int32) % N])
    dst = jnp.concatenate(
        [edge_index[1], N + (jnp.arange(pad, dtype=jnp.int32) % (ACC_ROWS - N))])
    srcp = src.reshape(E_PAD // EB, EB)
    dstp = dst.reshape(E_PAD // EB, EB)

    ea8 = edge_attr.reshape(E // 8, D)

    def blockdiag8(We):
        z = jnp.zeros((8, DE, 8, D), jnp.float32)
        z = z.at[jnp.arange(8), :, jnp.arange(8), :].set(We)
        return z.reshape(8 * DE, 8 * D)

    def tile8(be):
        return jnp.tile(be, (8,)).reshape(1, 8 * D)

    e0l, e0r = _edge_mlp1(ea8, blockdiag8(We0), tile8(be0))
    parts = _sc_aggregate(x, e0l, e0r, srcp, dstp)
    e1l, e1r, e2l, e2r = _edge_mlp2(ea8, blockdiag8(We1), tile8(be1),
                                    blockdiag8(We2), tile8(be2))
    h = _update(x, parts, Wn0, bn0.reshape(1, D))

    parts = _sc_aggregate(h, e1l, e1r, srcp, dstp)
    h = _update(h, parts, Wn1, bn1.reshape(1, D))

    parts = _sc_aggregate(h, e2l, e2r, srcp, dstp)
    return _update_pool(h, parts, Wn2, bn2.reshape(1, D), batch.reshape(N, 1),
                        W1, b1.reshape(1, D // 2), W2, b2.reshape(1, OUT))
